# Initial kernel scaffold; baseline (speedup 1.0000x reference)
import jax
import jax.numpy as jnp
from jax.experimental import pallas as pl


def kernel(feature_map, mask):
    raise NotImplementedError("write your pallas kernel here")


# speedup vs baseline: 11.2876x; 11.2876x over previous
import functools

import jax
import jax.numpy as jnp
from jax import lax
from jax.experimental import pallas as pl
from jax.experimental.pallas import tpu as pltpu
from jax.experimental.pallas import tpu_sc as plsc

N_ROWS = 100000
D_FEAT = 64
NUM_PAIRS = 1024
TO_PICK = 256

PAD_N = 100352
ROWS2D = PAD_N // 128
NC, NS, L = 2, 16, 16
SHARD = PAD_N // NS
NVEC = SHARD // L
ROWS_PER_SUB = NUM_PAIRS // NS


def _desc_key_u32(s):
  u = lax.bitcast_convert_type(s, jnp.uint32)
  sign = u >> jnp.uint32(31)
  flip = jnp.where(sign == jnp.uint32(1), jnp.uint32(0xFFFFFFFF),
                   jnp.uint32(0x80000000))
  return ~(u ^ flip)


def _kth_smallest_u32(v, k):
  def body(i, p):
    b = (jnp.uint32(31) - i.astype(jnp.uint32))
    cand = p | lax.shift_left(jnp.uint32(1), b)
    c = jnp.sum(jnp.where(v < cand, jnp.int32(1), jnp.int32(0)))
    return jnp.where(c >= k, p, cand)
  p = lax.fori_loop(0, 32, body, jnp.uint32(0))
  c_lt = jnp.sum(jnp.where(v < p, jnp.int32(1), jnp.int32(0)))
  return p, c_lt


def _select_kernel(mask_ref, g1_ref, g2_ref, vs1_ref, vs2_ref, prm_ref):
  mask = mask_ref[...]
  ridx = lax.broadcasted_iota(jnp.int32, (ROWS2D, 128), 0)
  cidx = lax.broadcasted_iota(jnp.int32, (ROWS2D, 128), 1)
  valid = (ridx * 128 + cidx) < N_ROWS

  w1 = 1.0 - mask
  w2 = mask
  s1sum = jnp.sum(jnp.where(valid, w1, 0.0))
  s2sum = jnp.sum(jnp.where(valid, w2, 0.0))
  s1 = g1_ref[...] + jnp.log(w1 / s1sum)
  s2 = g2_ref[...] + jnp.log(w2 / s2sum)

  pad_key = jnp.uint32(0xFFFFFFFF)
  v1 = jnp.where(valid, _desc_key_u32(s1), pad_key)
  v2 = jnp.where(valid, _desc_key_u32(s2), pad_key)

  t1, c1 = _kth_smallest_u32(v1, NUM_PAIRS)
  t2, c2 = _kth_smallest_u32(v2, NUM_PAIRS)

  sgn = jnp.uint32(0x80000000)
  vs1_ref[...] = lax.bitcast_convert_type(v1 ^ sgn, jnp.int32)
  vs2_ref[...] = lax.bitcast_convert_type(v2 ^ sgn, jnp.int32)

  t1s = lax.bitcast_convert_type(t1 ^ sgn, jnp.int32)
  t2s = lax.bitcast_convert_type(t2 ^ sgn, jnp.int32)
  li = lax.broadcasted_iota(jnp.int32, (1, 128), 1)
  prm = (jnp.where(li == 0, t1s, 0) + jnp.where(li == 1, c1, 0)
         + jnp.where(li == 2, t2s, 0) + jnp.where(li == 3, c2, 0))
  prm_ref[...] = prm


def _run_select(mask2d, g1_2d, g2_2d):
  return pl.pallas_call(
      _select_kernel,
      out_shape=(
          jax.ShapeDtypeStruct((ROWS2D, 128), jnp.int32),
          jax.ShapeDtypeStruct((ROWS2D, 128), jnp.int32),
          jax.ShapeDtypeStruct((1, 128), jnp.int32),
      ),
  )(mask2d, g1_2d, g2_2d)


def _sc_body(vs_hbm, prm_hbm, fmap_hbm, out_hbm,
             kv, ltbuf, eqbuf, vec16, asm, tmp, idxv, rows,
             cnt_sh, lt_sh, eq_sh, idx_sh, sem):
  cid = lax.axis_index("c")
  sid = lax.axis_index("s")
  lane = lax.broadcasted_iota(jnp.int32, (L,), 0)

  pltpu.sync_copy(prm_hbm, vec16)
  pv = vec16[...]

  def lane_at(vec, i):
    return jnp.sum(jnp.where(lane == i, vec, 0))

  thr = jnp.where(cid == 0, lane_at(pv, 0), lane_at(pv, 2))

  base = cid * PAD_N + sid * SHARD
  pltpu.sync_copy(vs_hbm.at[pl.ds(base, SHARD)], kv)
  gbase = sid * SHARD

  def step(j, carry):
    nlt, neq = carry
    k16 = kv[pl.ds(j * L, L)]
    gidx = lane + (gbase + j * L)
    m_lt = k16 < thr
    m_eq = k16 == thr
    one_lt = jnp.where(m_lt, jnp.int32(1), jnp.int32(0))
    one_eq = jnp.where(m_eq, jnp.int32(1), jnp.int32(0))
    plsc.store_scatter(ltbuf, [nlt + plsc.cumsum(one_lt) - 1], gidx, mask=m_lt)
    plsc.store_scatter(eqbuf, [neq + plsc.cumsum(one_eq) - 1], gidx, mask=m_eq)
    return nlt + jnp.sum(one_lt), neq + jnp.sum(one_eq)

  nlt, neq = lax.fori_loop(0, NVEC, step, (jnp.int32(0), jnp.int32(0)))

  pltpu.sync_copy(ltbuf, lt_sh.at[sid])
  pltpu.sync_copy(eqbuf, eq_sh.at[sid])
  cnt = jnp.where(lane == 0, nlt, 0) + jnp.where(lane == 1, neq, 0)
  vec16[...] = cnt
  pltpu.sync_copy(vec16, cnt_sh.at[sid])
  plsc.subcore_barrier()

  @pl.when(sid == 0)
  def _assemble():
    zeros = jnp.zeros((L,), jnp.int32)
    for z in range(NUM_PAIRS // L):
      asm[pl.ds(z * L, L)] = zeros

    def copy_list(r, off, take, sh):
      pltpu.sync_copy(sh.at[r], tmp)
      nv = (take + (L - 1)) >> 4

      def inner(j, off2):
        x = tmp[pl.ds(j * L, L)]
        rem = take - j * L
        m = lane < rem
        plsc.store_scatter(asm, [off2 + lane], x, mask=m)
        return off2 + jnp.minimum(rem, L)

      return lax.fori_loop(0, nv, inner, off)

    def lt_row(r, off):
      pltpu.sync_copy(cnt_sh.at[r], vec16)
      c = lane_at(vec16[...], 0)
      return copy_list(r, off, c, lt_sh)

    off = lax.fori_loop(0, NS, lt_row, jnp.int32(0))

    def eq_row(r, off):
      pltpu.sync_copy(cnt_sh.at[r], vec16)
      c = lane_at(vec16[...], 1)
      take = jnp.minimum(c, jnp.int32(NUM_PAIRS) - off)
      return copy_list(r, off, take, eq_sh)

    lax.fori_loop(0, NS, eq_row, off)
    pltpu.sync_copy(asm, idx_sh)

  plsc.subcore_barrier()

  pltpu.sync_copy(idx_sh.at[pl.ds(sid * ROWS_PER_SUB, ROWS_PER_SUB)], idxv)
  pltpu.async_copy(fmap_hbm.at[idxv], rows, sem).wait()
  out_base = cid * NUM_PAIRS + sid * ROWS_PER_SUB
  pltpu.sync_copy(rows, out_hbm.at[pl.ds(out_base, ROWS_PER_SUB)])


def _run_gather(vs_flat, prm16, feature_map):
  mesh = plsc.VectorSubcoreMesh(core_axis_name="c", subcore_axis_name="s",
                                num_cores=NC, num_subcores=NS)
  fn = pl.kernel(
      _sc_body,
      out_type=jax.ShapeDtypeStruct((2 * NUM_PAIRS, D_FEAT), jnp.float32),
      mesh=mesh,
      scratch_types=[
          pltpu.VMEM((SHARD,), jnp.int32),
          pltpu.VMEM((SHARD,), jnp.int32),
          pltpu.VMEM((SHARD,), jnp.int32),
          pltpu.VMEM((L,), jnp.int32),
          pltpu.VMEM((NUM_PAIRS,), jnp.int32),
          pltpu.VMEM((SHARD,), jnp.int32),
          pltpu.VMEM((ROWS_PER_SUB,), jnp.int32),
          pltpu.VMEM((ROWS_PER_SUB, D_FEAT), jnp.float32),
          pltpu.VMEM_SHARED((NS, L), jnp.int32),
          pltpu.VMEM_SHARED((NS, SHARD), jnp.int32),
          pltpu.VMEM_SHARED((NS, SHARD), jnp.int32),
          pltpu.VMEM_SHARED((NUM_PAIRS,), jnp.int32),
          pltpu.SemaphoreType.DMA,
      ],
      compiler_params=pltpu.CompilerParams(needs_layout_passes=False,
                                           use_tc_tiling_on_sc=False),
  )
  return fn(vs_flat, prm16, feature_map)


def _topk_mean_kernel(p_ref, out_ref):
  c1 = p_ref[0]
  c2 = p_ref[1]
  n1 = jnp.sum(c1 * c1, axis=1, keepdims=True)
  n2 = jnp.sum(c2 * c2, axis=1, keepdims=True).reshape(1, NUM_PAIRS)
  g = lax.dot_general(c1, c2, (((1,), (1,)), ((), ())),
                      preferred_element_type=jnp.float32)
  d2 = jnp.maximum(n1 + n2 - 2.0 * g, 0.0)
  u = lax.bitcast_convert_type(d2, jnp.uint32) & jnp.uint32(0x7FFFFFFF)
  v = ~u
  p, c_lt = _kth_smallest_u32(v, TO_PICK)
  thr_d2 = lax.bitcast_convert_type(~p, jnp.float32)
  ssum = jnp.sum(jnp.where(v < p, jnp.sqrt(d2), 0.0))
  total = ssum + (jnp.int32(TO_PICK) - c_lt).astype(jnp.float32) * jnp.sqrt(thr_d2)
  out_ref[...] = jnp.full((1, 1), 0.0) + total * (1.0 / TO_PICK)


def _run_topk_mean(picked):
  return pl.pallas_call(
      _topk_mean_kernel,
      out_shape=jax.ShapeDtypeStruct((1, 1), jnp.float32),
  )(picked.reshape(2, NUM_PAIRS, D_FEAT))


def kernel(feature_map, mask):
  n = feature_map.shape[0]
  kc = jax.random.key(42)
  g1 = jax.random.gumbel(jax.random.fold_in(kc, 0), (n,), jnp.float32)
  g2 = jax.random.gumbel(jax.random.fold_in(kc, 1), (n,), jnp.float32)
  pad = PAD_N - n
  mask2d = jnp.pad(mask, (0, pad)).reshape(ROWS2D, 128)
  g1_2d = jnp.pad(g1, (0, pad)).reshape(ROWS2D, 128)
  g2_2d = jnp.pad(g2, (0, pad)).reshape(ROWS2D, 128)

  vs1, vs2, prm = _run_select(mask2d, g1_2d, g2_2d)
  vs_flat = jnp.concatenate([vs1.reshape(-1), vs2.reshape(-1)])
  prm16 = prm.reshape(-1)[:L]

  picked = _run_gather(vs_flat, prm16, feature_map)
  out = _run_topk_mean(picked)
  return out[0, 0]


# speedup vs baseline: 12.6668x; 1.1222x over previous
import functools

import jax
import jax.numpy as jnp
from jax import lax
from jax.experimental import pallas as pl
from jax.experimental.pallas import tpu as pltpu
from jax.experimental.pallas import tpu_sc as plsc

N_ROWS = 100000
D_FEAT = 64
NUM_PAIRS = 1024
TO_PICK = 256

PAD_N = 100352
ROWS2D = PAD_N // 128
NC, NS, L = 2, 16, 16
SHARD = PAD_N // NS
NVEC = SHARD // L
ROWS_PER_SUB = NUM_PAIRS // NS


def _desc_key_u32(s):
  u = lax.bitcast_convert_type(s, jnp.uint32)
  sign = u >> jnp.uint32(31)
  flip = jnp.where(sign == jnp.uint32(1), jnp.uint32(0xFFFFFFFF),
                   jnp.uint32(0x80000000))
  return ~(u ^ flip)


def _kth_smallest_u32(v, k):
  def count_lt(cand):
    return jnp.sum(jnp.where(v < cand, jnp.int32(1), jnp.int32(0)))
  def body(i, p):
    sh = (jnp.uint32(30) - 2 * i.astype(jnp.uint32))
    q = lax.shift_left(jnp.uint32(1), sh)
    c1m, c2m, c3m = p | q, p | (q + q), p | (q + q + q)
    n1, n2, n3 = count_lt(c1m), count_lt(c2m), count_lt(c3m)
    return jnp.where(n1 >= k, p,
                     jnp.where(n2 >= k, c1m,
                               jnp.where(n3 >= k, c2m, c3m)))
  p = lax.fori_loop(0, 16, body, jnp.uint32(0))
  c_lt = jnp.sum(jnp.where(v < p, jnp.int32(1), jnp.int32(0)))
  return p, c_lt


def _select_kernel(mask_ref, g1_ref, g2_ref, vs1_ref, vs2_ref, prm_ref):
  mask = mask_ref[...]
  ridx = lax.broadcasted_iota(jnp.int32, (ROWS2D, 128), 0)
  cidx = lax.broadcasted_iota(jnp.int32, (ROWS2D, 128), 1)
  valid = (ridx * 128 + cidx) < N_ROWS

  w1 = 1.0 - mask
  w2 = mask
  s1sum = jnp.sum(jnp.where(valid, w1, 0.0))
  s2sum = jnp.sum(jnp.where(valid, w2, 0.0))
  s1 = g1_ref[...] + jnp.log(w1 / s1sum)
  s2 = g2_ref[...] + jnp.log(w2 / s2sum)

  pad_key = jnp.uint32(0xFFFFFFFF)
  v1 = jnp.where(valid, _desc_key_u32(s1), pad_key)
  v2 = jnp.where(valid, _desc_key_u32(s2), pad_key)

  t1, c1 = _kth_smallest_u32(v1, NUM_PAIRS)
  t2, c2 = _kth_smallest_u32(v2, NUM_PAIRS)

  sgn = jnp.uint32(0x80000000)
  vs1_ref[...] = lax.bitcast_convert_type(v1 ^ sgn, jnp.int32)
  vs2_ref[...] = lax.bitcast_convert_type(v2 ^ sgn, jnp.int32)

  t1s = lax.bitcast_convert_type(t1 ^ sgn, jnp.int32)
  t2s = lax.bitcast_convert_type(t2 ^ sgn, jnp.int32)
  li = lax.broadcasted_iota(jnp.int32, (1, 128), 1)
  prm = (jnp.where(li == 0, t1s, 0) + jnp.where(li == 1, c1, 0)
         + jnp.where(li == 2, t2s, 0) + jnp.where(li == 3, c2, 0))
  prm_ref[...] = prm


def _run_select(mask2d, g1_2d, g2_2d):
  return pl.pallas_call(
      _select_kernel,
      out_shape=(
          jax.ShapeDtypeStruct((ROWS2D, 128), jnp.int32),
          jax.ShapeDtypeStruct((ROWS2D, 128), jnp.int32),
          jax.ShapeDtypeStruct((1, 128), jnp.int32),
      ),
  )(mask2d, g1_2d, g2_2d)


def _sc_body(vs_hbm, prm_hbm, fmap_hbm, wide_hbm, par_hbm,
             kv, ltbuf, eqbuf, vec16, asmh, aspar, tmp, idxv, rows,
             cnt_sh, lt_sh, eq_sh, idxh_sh, sem):
  cid = lax.axis_index("c")
  sid = lax.axis_index("s")
  lane = lax.broadcasted_iota(jnp.int32, (L,), 0)

  pltpu.sync_copy(prm_hbm, vec16)
  pv = vec16[...]

  def lane_at(vec, i):
    return jnp.sum(jnp.where(lane == i, vec, 0))

  thr = jnp.where(cid == 0, lane_at(pv, 0), lane_at(pv, 2))

  base = cid * PAD_N + sid * SHARD
  pltpu.sync_copy(vs_hbm.at[pl.ds(base, SHARD)], kv)
  gbase = sid * SHARD

  def step(j, carry):
    nlt, neq = carry
    k16 = kv[pl.ds(j * L, L)]
    gidx = lane + (gbase + j * L)
    m_lt = k16 < thr
    m_eq = k16 == thr
    one_lt = jnp.where(m_lt, jnp.int32(1), jnp.int32(0))
    one_eq = jnp.where(m_eq, jnp.int32(1), jnp.int32(0))
    plsc.store_scatter(ltbuf, [nlt + plsc.cumsum(one_lt) - 1], gidx, mask=m_lt)
    plsc.store_scatter(eqbuf, [neq + plsc.cumsum(one_eq) - 1], gidx, mask=m_eq)
    return nlt + jnp.sum(one_lt), neq + jnp.sum(one_eq)

  nlt, neq = lax.fori_loop(0, NVEC, step, (jnp.int32(0), jnp.int32(0)))

  pltpu.sync_copy(ltbuf, lt_sh.at[sid])
  pltpu.sync_copy(eqbuf, eq_sh.at[sid])
  cnt = jnp.where(lane == 0, nlt, 0) + jnp.where(lane == 1, neq, 0)
  vec16[...] = cnt
  pltpu.sync_copy(vec16, cnt_sh.at[sid])
  plsc.subcore_barrier()

  @pl.when(sid == 0)
  def _assemble():
    zeros = jnp.zeros((L,), jnp.int32)
    for z in range(NUM_PAIRS // L):
      asmh[pl.ds(z * L, L)] = zeros
      aspar[pl.ds(z * L, L)] = zeros

    def copy_list(r, off, take, sh):
      pltpu.sync_copy(sh.at[r], tmp)
      nv = (take + (L - 1)) >> 4

      def inner(j, off2):
        x = tmp[pl.ds(j * L, L)]
        rem = take - j * L
        m = lane < rem
        pos = [off2 + lane]
        plsc.store_scatter(asmh, pos, lax.shift_right_logical(x, 1), mask=m)
        plsc.store_scatter(aspar, pos, x & 1, mask=m)
        return off2 + jnp.minimum(rem, L)

      return lax.fori_loop(0, nv, inner, off)

    def lt_row(r, off):
      pltpu.sync_copy(cnt_sh.at[r], vec16)
      c = lane_at(vec16[...], 0)
      return copy_list(r, off, c, lt_sh)

    off = lax.fori_loop(0, NS, lt_row, jnp.int32(0))

    def eq_row(r, off):
      pltpu.sync_copy(cnt_sh.at[r], vec16)
      c = lane_at(vec16[...], 1)
      take = jnp.minimum(c, jnp.int32(NUM_PAIRS) - off)
      return copy_list(r, off, take, eq_sh)

    lax.fori_loop(0, NS, eq_row, off)
    pltpu.sync_copy(asmh, idxh_sh)
    pltpu.sync_copy(aspar, par_hbm.at[pl.ds(cid * NUM_PAIRS, NUM_PAIRS)])

  plsc.subcore_barrier()

  pltpu.sync_copy(idxh_sh.at[pl.ds(sid * ROWS_PER_SUB, ROWS_PER_SUB)], idxv)
  pltpu.async_copy(fmap_hbm.at[idxv], rows, sem).wait()
  out_base = cid * NUM_PAIRS + sid * ROWS_PER_SUB
  pltpu.sync_copy(rows, wide_hbm.at[pl.ds(out_base, ROWS_PER_SUB)])


def _run_gather(vs_flat, prm16, feature_map):
  mesh = plsc.VectorSubcoreMesh(core_axis_name="c", subcore_axis_name="s",
                                num_cores=NC, num_subcores=NS)
  fn = pl.kernel(
      _sc_body,
      out_type=(
          jax.ShapeDtypeStruct((2 * NUM_PAIRS, 128), jnp.float32),
          jax.ShapeDtypeStruct((2 * NUM_PAIRS,), jnp.int32),
      ),
      mesh=mesh,
      scratch_types=[
          pltpu.VMEM((SHARD,), jnp.int32),
          pltpu.VMEM((SHARD,), jnp.int32),
          pltpu.VMEM((SHARD,), jnp.int32),
          pltpu.VMEM((L,), jnp.int32),
          pltpu.VMEM((NUM_PAIRS,), jnp.int32),
          pltpu.VMEM((NUM_PAIRS,), jnp.int32),
          pltpu.VMEM((SHARD,), jnp.int32),
          pltpu.VMEM((ROWS_PER_SUB,), jnp.int32),
          pltpu.VMEM((ROWS_PER_SUB, 128), jnp.float32),
          pltpu.VMEM_SHARED((NS, L), jnp.int32),
          pltpu.VMEM_SHARED((NS, SHARD), jnp.int32),
          pltpu.VMEM_SHARED((NS, SHARD), jnp.int32),
          pltpu.VMEM_SHARED((NUM_PAIRS,), jnp.int32),
          pltpu.SemaphoreType.DMA,
      ],
      compiler_params=pltpu.CompilerParams(needs_layout_passes=False,
                                           use_tc_tiling_on_sc=False),
  )
  return fn(vs_flat, prm16, feature_map)


def _topk_mean_kernel(w_ref, par_ref, out_ref):
  w1 = w_ref[0]
  w2 = w_ref[1]
  p1 = par_ref[0]
  p2 = par_ref[1]
  c1 = w1[:, 0:D_FEAT] * (1.0 - p1) + w1[:, D_FEAT:128] * p1
  c2 = w2[:, 0:D_FEAT] * (1.0 - p2) + w2[:, D_FEAT:128] * p2
  n1 = jnp.sum(c1 * c1, axis=1, keepdims=True)
  n2 = jnp.sum(c2 * c2, axis=1, keepdims=True).reshape(1, NUM_PAIRS)
  g = lax.dot_general(c1, c2, (((1,), (1,)), ((), ())),
                      preferred_element_type=jnp.float32)
  d2 = jnp.maximum(n1 + n2 - 2.0 * g, 0.0)
  u = lax.bitcast_convert_type(d2, jnp.uint32) & jnp.uint32(0x7FFFFFFF)
  v = ~u
  p, c_lt = _kth_smallest_u32(v, TO_PICK)
  thr_d2 = lax.bitcast_convert_type(~p, jnp.float32)
  ssum = jnp.sum(jnp.where(v < p, jnp.sqrt(d2), 0.0))
  total = ssum + (jnp.int32(TO_PICK) - c_lt).astype(jnp.float32) * jnp.sqrt(thr_d2)
  out_ref[...] = jnp.full((1, 1), 0.0) + total * (1.0 / TO_PICK)


def _run_topk_mean(wide, parf):
  return pl.pallas_call(
      _topk_mean_kernel,
      out_shape=jax.ShapeDtypeStruct((1, 1), jnp.float32),
  )(wide.reshape(2, NUM_PAIRS, 128), parf)


def kernel(feature_map, mask):
  n = feature_map.shape[0]
  kc = jax.random.key(42)
  g1 = jax.random.gumbel(jax.random.fold_in(kc, 0), (n,), jnp.float32)
  g2 = jax.random.gumbel(jax.random.fold_in(kc, 1), (n,), jnp.float32)
  pad = PAD_N - n
  mask2d = jnp.pad(mask, (0, pad)).reshape(ROWS2D, 128)
  g1_2d = jnp.pad(g1, (0, pad)).reshape(ROWS2D, 128)
  g2_2d = jnp.pad(g2, (0, pad)).reshape(ROWS2D, 128)

  vs1, vs2, prm = _run_select(mask2d, g1_2d, g2_2d)
  vs_flat = jnp.concatenate([vs1.reshape(-1), vs2.reshape(-1)])
  prm16 = prm.reshape(-1)[:L]

  fmap_pairs = feature_map.reshape(N_ROWS // 2, 2 * D_FEAT)
  wide, par = _run_gather(vs_flat, prm16, fmap_pairs)
  parf = par.astype(jnp.float32).reshape(2, NUM_PAIRS, 1)
  out = _run_topk_mean(wide, parf)
  return out[0, 0]


# speedup vs baseline: 15.1021x; 1.1923x over previous
import functools

import jax
import jax.numpy as jnp
from jax import lax
from jax.experimental import pallas as pl
from jax.experimental.pallas import tpu as pltpu
from jax.experimental.pallas import tpu_sc as plsc

N_ROWS = 100000
D_FEAT = 64
NUM_PAIRS = 1024
TO_PICK = 256

PAD_N = 100352
ROWS2D = PAD_N // 128
NC, NS, L = 2, 16, 16
SHARD = PAD_N // NS
NVEC = SHARD // L
ROWS_PER_SUB = NUM_PAIRS // NS


def _desc_key_u32(s):
  u = lax.bitcast_convert_type(s, jnp.uint32)
  sign = u >> jnp.uint32(31)
  flip = jnp.where(sign == jnp.uint32(1), jnp.uint32(0xFFFFFFFF),
                   jnp.uint32(0x80000000))
  return ~(u ^ flip)


def _kth_smallest_u32(v, k):
  def count_lt(cand):
    return jnp.sum(jnp.where(v < cand, jnp.int32(1), jnp.int32(0)))
  def body(i, p):
    sh = (jnp.uint32(30) - 2 * i.astype(jnp.uint32))
    q = lax.shift_left(jnp.uint32(1), sh)
    c1m, c2m, c3m = p | q, p | (q + q), p | (q + q + q)
    n1, n2, n3 = count_lt(c1m), count_lt(c2m), count_lt(c3m)
    return jnp.where(n1 >= k, p,
                     jnp.where(n2 >= k, c1m,
                               jnp.where(n3 >= k, c2m, c3m)))
  p = lax.fori_loop(0, 16, body, jnp.uint32(0))
  c_lt = jnp.sum(jnp.where(v < p, jnp.int32(1), jnp.int32(0)))
  return p, c_lt


def _select_kernel(mask_ref, g1_ref, g2_ref, vs1_ref, vs2_ref, prm_ref):
  mask = mask_ref[...]
  ridx = lax.broadcasted_iota(jnp.int32, (ROWS2D, 128), 0)
  cidx = lax.broadcasted_iota(jnp.int32, (ROWS2D, 128), 1)
  valid = (ridx * 128 + cidx) < N_ROWS

  w1 = 1.0 - mask
  w2 = mask
  s1sum = jnp.sum(jnp.where(valid, w1, 0.0))
  s2sum = jnp.sum(jnp.where(valid, w2, 0.0))
  s1 = g1_ref[...] + jnp.log(w1 / s1sum)
  s2 = g2_ref[...] + jnp.log(w2 / s2sum)

  pad_key = jnp.uint32(0xFFFFFFFF)
  v1 = jnp.where(valid, _desc_key_u32(s1), pad_key)
  v2 = jnp.where(valid, _desc_key_u32(s2), pad_key)

  t1, c1 = _kth_smallest_u32(v1, NUM_PAIRS)
  t2, c2 = _kth_smallest_u32(v2, NUM_PAIRS)

  sgn = jnp.uint32(0x80000000)
  vs1_ref[...] = lax.bitcast_convert_type(v1 ^ sgn, jnp.int32)
  vs2_ref[...] = lax.bitcast_convert_type(v2 ^ sgn, jnp.int32)

  t1s = lax.bitcast_convert_type(t1 ^ sgn, jnp.int32)
  t2s = lax.bitcast_convert_type(t2 ^ sgn, jnp.int32)
  li = lax.broadcasted_iota(jnp.int32, (1, 128), 1)
  prm = (jnp.where(li == 0, t1s, 0) + jnp.where(li == 1, c1, 0)
         + jnp.where(li == 2, t2s, 0) + jnp.where(li == 3, c2, 0))
  prm_ref[...] = prm


def _run_select(mask2d, g1_2d, g2_2d):
  return pl.pallas_call(
      _select_kernel,
      out_shape=(
          jax.ShapeDtypeStruct((ROWS2D, 128), jnp.int32),
          jax.ShapeDtypeStruct((ROWS2D, 128), jnp.int32),
          jax.ShapeDtypeStruct((1, 128), jnp.int32),
      ),
  )(mask2d, g1_2d, g2_2d)


def _sc_compact_body(vs_hbm, prm_hbm, idxh_hbm, par_hbm,
                     kv, ltbuf, eqbuf, vec16, asmh, aspar, tmp,
                     cnt_sh, lt_sh, eq_sh):
  cid = lax.axis_index("c")
  sid = lax.axis_index("s")
  lane = lax.broadcasted_iota(jnp.int32, (L,), 0)

  pltpu.sync_copy(prm_hbm, vec16)
  pv = vec16[...]

  def lane_at(vec, i):
    return jnp.sum(jnp.where(lane == i, vec, 0))

  thr = jnp.where(cid == 0, lane_at(pv, 0), lane_at(pv, 2))

  base = cid * PAD_N + sid * SHARD
  pltpu.sync_copy(vs_hbm.at[pl.ds(base, SHARD)], kv)
  gbase = sid * SHARD

  def step(j, carry):
    nlt, neq = carry
    k16 = kv[pl.ds(j * L, L)]
    gidx = lane + (gbase + j * L)
    m_lt = k16 < thr
    m_eq = k16 == thr
    one_lt = jnp.where(m_lt, jnp.int32(1), jnp.int32(0))
    one_eq = jnp.where(m_eq, jnp.int32(1), jnp.int32(0))
    plsc.store_scatter(ltbuf, [nlt + plsc.cumsum(one_lt) - 1], gidx, mask=m_lt)
    plsc.store_scatter(eqbuf, [neq + plsc.cumsum(one_eq) - 1], gidx, mask=m_eq)
    return nlt + jnp.sum(one_lt), neq + jnp.sum(one_eq)

  nlt, neq = lax.fori_loop(0, NVEC, step, (jnp.int32(0), jnp.int32(0)))

  pltpu.sync_copy(ltbuf, lt_sh.at[sid])
  pltpu.sync_copy(eqbuf, eq_sh.at[sid])
  cnt = jnp.where(lane == 0, nlt, 0) + jnp.where(lane == 1, neq, 0)
  vec16[...] = cnt
  pltpu.sync_copy(vec16, cnt_sh.at[sid])
  plsc.subcore_barrier()

  @pl.when(sid == 0)
  def _assemble():
    zeros = jnp.zeros((L,), jnp.int32)
    for z in range(NUM_PAIRS // L):
      asmh[pl.ds(z * L, L)] = zeros
      aspar[pl.ds(z * L, L)] = zeros

    def copy_list(r, off, take, sh):
      pltpu.sync_copy(sh.at[r], tmp)
      nv = (take + (L - 1)) >> 4

      def inner(j, off2):
        x = tmp[pl.ds(j * L, L)]
        rem = take - j * L
        m = lane < rem
        pos = [off2 + lane]
        plsc.store_scatter(asmh, pos, lax.shift_right_logical(x, 1), mask=m)
        plsc.store_scatter(aspar, pos, x & 1, mask=m)
        return off2 + jnp.minimum(rem, L)

      return lax.fori_loop(0, nv, inner, off)

    def lt_row(r, off):
      pltpu.sync_copy(cnt_sh.at[r], vec16)
      c = lane_at(vec16[...], 0)
      return copy_list(r, off, c, lt_sh)

    off = lax.fori_loop(0, NS, lt_row, jnp.int32(0))

    def eq_row(r, off):
      pltpu.sync_copy(cnt_sh.at[r], vec16)
      c = lane_at(vec16[...], 1)
      take = jnp.minimum(c, jnp.int32(NUM_PAIRS) - off)
      return copy_list(r, off, take, eq_sh)

    lax.fori_loop(0, NS, eq_row, off)
    pltpu.sync_copy(asmh, idxh_hbm.at[pl.ds(cid * NUM_PAIRS, NUM_PAIRS)])
    pltpu.sync_copy(aspar, par_hbm.at[pl.ds(cid * NUM_PAIRS, NUM_PAIRS)])


def _sc_gather_body(idxh_hbm, fmap_hbm, wide_hbm, idxv, rows, sem):
  cid = lax.axis_index("c")
  sid = lax.axis_index("s")
  base = cid * NUM_PAIRS + sid * ROWS_PER_SUB
  pltpu.sync_copy(idxh_hbm.at[pl.ds(base, ROWS_PER_SUB)], idxv)
  pltpu.async_copy(fmap_hbm.at[idxv], rows, sem).wait()
  pltpu.sync_copy(rows, wide_hbm.at[pl.ds(base, ROWS_PER_SUB)])


def _run_compact(vs_flat, prm16):
  mesh = plsc.VectorSubcoreMesh(core_axis_name="c", subcore_axis_name="s",
                                num_cores=NC, num_subcores=NS)
  fn = pl.kernel(
      _sc_compact_body,
      out_type=(
          jax.ShapeDtypeStruct((2 * NUM_PAIRS,), jnp.int32),
          jax.ShapeDtypeStruct((2 * NUM_PAIRS,), jnp.int32),
      ),
      mesh=mesh,
      scratch_types=[
          pltpu.VMEM((SHARD,), jnp.int32),
          pltpu.VMEM((SHARD,), jnp.int32),
          pltpu.VMEM((SHARD,), jnp.int32),
          pltpu.VMEM((L,), jnp.int32),
          pltpu.VMEM((NUM_PAIRS,), jnp.int32),
          pltpu.VMEM((NUM_PAIRS,), jnp.int32),
          pltpu.VMEM((SHARD,), jnp.int32),
          pltpu.VMEM_SHARED((NS, L), jnp.int32),
          pltpu.VMEM_SHARED((NS, SHARD), jnp.int32),
          pltpu.VMEM_SHARED((NS, SHARD), jnp.int32),
      ],
      compiler_params=pltpu.CompilerParams(needs_layout_passes=False,
                                           use_tc_tiling_on_sc=False),
  )
  return fn(vs_flat, prm16)


def _run_gather(idxh, fmap_pairs):
  mesh = plsc.VectorSubcoreMesh(core_axis_name="c", subcore_axis_name="s",
                                num_cores=NC, num_subcores=NS)
  fn = pl.kernel(
      _sc_gather_body,
      out_type=jax.ShapeDtypeStruct((2 * NUM_PAIRS, 128), jnp.float32),
      mesh=mesh,
      scratch_types=[
          pltpu.VMEM((ROWS_PER_SUB,), jnp.int32),
          pltpu.VMEM((ROWS_PER_SUB, 128), jnp.float32),
          pltpu.SemaphoreType.DMA,
      ],
      compiler_params=pltpu.CompilerParams(needs_layout_passes=False,
                                           use_tc_tiling_on_sc=True),
  )
  return fn(idxh, fmap_pairs)


def _topk_mean_kernel(w_ref, par_ref, out_ref):
  w1 = w_ref[0]
  w2 = w_ref[1]
  p1 = par_ref[0]
  p2 = par_ref[1]
  c1 = w1[:, 0:D_FEAT] * (1.0 - p1) + w1[:, D_FEAT:128] * p1
  c2 = w2[:, 0:D_FEAT] * (1.0 - p2) + w2[:, D_FEAT:128] * p2
  n1 = jnp.sum(c1 * c1, axis=1, keepdims=True)
  n2 = jnp.sum(c2 * c2, axis=1, keepdims=True).reshape(1, NUM_PAIRS)
  g = lax.dot_general(c1, c2, (((1,), (1,)), ((), ())),
                      preferred_element_type=jnp.float32)
  d2 = jnp.maximum(n1 + n2 - 2.0 * g, 0.0)
  u = lax.bitcast_convert_type(d2, jnp.uint32) & jnp.uint32(0x7FFFFFFF)
  v = ~u
  p, c_lt = _kth_smallest_u32(v, TO_PICK)
  thr_d2 = lax.bitcast_convert_type(~p, jnp.float32)
  ssum = jnp.sum(jnp.where(v < p, jnp.sqrt(d2), 0.0))
  total = ssum + (jnp.int32(TO_PICK) - c_lt).astype(jnp.float32) * jnp.sqrt(thr_d2)
  out_ref[...] = jnp.full((1, 1), 0.0) + total * (1.0 / TO_PICK)


def _run_topk_mean(wide, parf):
  return pl.pallas_call(
      _topk_mean_kernel,
      out_shape=jax.ShapeDtypeStruct((1, 1), jnp.float32),
  )(wide.reshape(2, NUM_PAIRS, 128), parf)


def kernel(feature_map, mask):
  n = feature_map.shape[0]
  kc = jax.random.key(42)
  g1 = jax.random.gumbel(jax.random.fold_in(kc, 0), (n,), jnp.float32)
  g2 = jax.random.gumbel(jax.random.fold_in(kc, 1), (n,), jnp.float32)
  pad = PAD_N - n
  mask2d = jnp.pad(mask, (0, pad)).reshape(ROWS2D, 128)
  g1_2d = jnp.pad(g1, (0, pad)).reshape(ROWS2D, 128)
  g2_2d = jnp.pad(g2, (0, pad)).reshape(ROWS2D, 128)

  vs1, vs2, prm = _run_select(mask2d, g1_2d, g2_2d)
  vs_flat = jnp.concatenate([vs1.reshape(-1), vs2.reshape(-1)])
  prm16 = prm.reshape(-1)[:L]

  fmap_pairs = feature_map.reshape(N_ROWS // 2, 2 * D_FEAT)
  idxh, par = _run_compact(vs_flat, prm16)
  wide = _run_gather(idxh, fmap_pairs)
  parf = par.astype(jnp.float32).reshape(2, NUM_PAIRS, 1)
  out = _run_topk_mean(wide, parf)
  return out[0, 0]


# speedup vs baseline: 16.4109x; 1.0867x over previous
import functools

import jax
import jax.numpy as jnp
from jax import lax
from jax.experimental import pallas as pl
from jax.experimental.pallas import tpu as pltpu
from jax.experimental.pallas import tpu_sc as plsc

N_ROWS = 100000
D_FEAT = 64
NUM_PAIRS = 1024
TO_PICK = 256

PAD_N = 100352
ROWS2D = PAD_N // 128
NC, NS, L = 2, 16, 16
SHARD = PAD_N // NS
NVEC = SHARD // L
ROWS_PER_SUB = NUM_PAIRS // NS


def _desc_key_u32(s):
  u = lax.bitcast_convert_type(s, jnp.uint32)
  sign = u >> jnp.uint32(31)
  flip = jnp.where(sign == jnp.uint32(1), jnp.uint32(0xFFFFFFFF),
                   jnp.uint32(0x80000000))
  return ~(u ^ flip)


def _kth_smallest_u32(v, k):
  def count_lt(cand):
    return jnp.sum(jnp.where(v < cand, jnp.int32(1), jnp.int32(0)))
  def body(i, p):
    sh = (jnp.uint32(30) - 2 * i.astype(jnp.uint32))
    q = lax.shift_left(jnp.uint32(1), sh)
    c1m, c2m, c3m = p | q, p | (q + q), p | (q + q + q)
    n1, n2, n3 = count_lt(c1m), count_lt(c2m), count_lt(c3m)
    return jnp.where(n1 >= k, p,
                     jnp.where(n2 >= k, c1m,
                               jnp.where(n3 >= k, c2m, c3m)))
  p = lax.fori_loop(0, 16, body, jnp.uint32(0))
  c_lt = jnp.sum(jnp.where(v < p, jnp.int32(1), jnp.int32(0)))
  return p, c_lt


def _select_kernel(mask_ref, g1_ref, g2_ref, vs1_ref, vs2_ref, prm_ref):
  mask = mask_ref[...]
  ridx = lax.broadcasted_iota(jnp.int32, (ROWS2D, 128), 0)
  cidx = lax.broadcasted_iota(jnp.int32, (ROWS2D, 128), 1)
  valid = (ridx * 128 + cidx) < N_ROWS

  w1 = 1.0 - mask
  w2 = mask
  s1sum = jnp.sum(jnp.where(valid, w1, 0.0))
  s2sum = jnp.sum(jnp.where(valid, w2, 0.0))
  s1 = g1_ref[...] + jnp.log(w1 / s1sum)
  s2 = g2_ref[...] + jnp.log(w2 / s2sum)

  pad_key = jnp.uint32(0xFFFFFFFF)
  v1 = jnp.where(valid, _desc_key_u32(s1), pad_key)
  v2 = jnp.where(valid, _desc_key_u32(s2), pad_key)

  t1, c1 = _kth_smallest_u32(v1, NUM_PAIRS)
  t2, c2 = _kth_smallest_u32(v2, NUM_PAIRS)

  sgn = jnp.uint32(0x80000000)
  vs1_ref[...] = lax.bitcast_convert_type(v1 ^ sgn, jnp.int32)
  vs2_ref[...] = lax.bitcast_convert_type(v2 ^ sgn, jnp.int32)

  t1s = lax.bitcast_convert_type(t1 ^ sgn, jnp.int32)
  t2s = lax.bitcast_convert_type(t2 ^ sgn, jnp.int32)
  li = lax.broadcasted_iota(jnp.int32, (1, 128), 1)
  prm = (jnp.where(li == 0, t1s, 0) + jnp.where(li == 1, c1, 0)
         + jnp.where(li == 2, t2s, 0) + jnp.where(li == 3, c2, 0))
  prm_ref[...] = prm


def _run_select(mask2d, g1_2d, g2_2d):
  return pl.pallas_call(
      _select_kernel,
      out_shape=(
          jax.ShapeDtypeStruct((ROWS2D, 128), jnp.int32),
          jax.ShapeDtypeStruct((ROWS2D, 128), jnp.int32),
          jax.ShapeDtypeStruct((1, 128), jnp.int32),
      ),
  )(mask2d, g1_2d, g2_2d)


def _sc_compact_body(vs_hbm, prm_hbm, idxh_hbm, par_hbm,
                     kv, ltbuf, eqbuf, vec16, asmh, aspar, tmp,
                     cnt_sh, lt_sh, eq_sh):
  cid = lax.axis_index("c")
  sid = lax.axis_index("s")
  lane = lax.broadcasted_iota(jnp.int32, (L,), 0)

  pltpu.sync_copy(prm_hbm, vec16)
  pv = vec16[...]

  def lane_at(vec, i):
    return jnp.sum(jnp.where(lane == i, vec, 0))

  thr = jnp.where(cid == 0, lane_at(pv, 0), lane_at(pv, 2))

  base = cid * PAD_N + sid * SHARD
  pltpu.sync_copy(vs_hbm.at[pl.ds(base, SHARD)], kv)
  gbase = sid * SHARD

  def step(j, carry):
    nlt, neq = carry
    k16 = kv[pl.ds(j * L, L)]
    gidx = lane + (gbase + j * L)
    m_lt = k16 < thr
    m_eq = k16 == thr
    one_lt = jnp.where(m_lt, jnp.int32(1), jnp.int32(0))
    one_eq = jnp.where(m_eq, jnp.int32(1), jnp.int32(0))
    plsc.store_scatter(ltbuf, [nlt + plsc.cumsum(one_lt) - 1], gidx, mask=m_lt)
    plsc.store_scatter(eqbuf, [neq + plsc.cumsum(one_eq) - 1], gidx, mask=m_eq)
    return nlt + jnp.sum(one_lt), neq + jnp.sum(one_eq)

  nlt, neq = lax.fori_loop(0, NVEC, step, (jnp.int32(0), jnp.int32(0)))

  pltpu.sync_copy(ltbuf, lt_sh.at[sid])
  pltpu.sync_copy(eqbuf, eq_sh.at[sid])
  cnt = jnp.where(lane == 0, nlt, 0) + jnp.where(lane == 1, neq, 0)
  vec16[...] = cnt
  pltpu.sync_copy(vec16, cnt_sh.at[sid])
  plsc.subcore_barrier()

  @pl.when(sid == 0)
  def _assemble():
    zeros = jnp.zeros((L,), jnp.int32)
    for z in range(NUM_PAIRS // L):
      asmh[pl.ds(z * L, L)] = zeros
      aspar[pl.ds(z * L, L)] = zeros

    def copy_list(r, off, take, sh):
      pltpu.sync_copy(sh.at[r], tmp)
      nv = (take + (L - 1)) >> 4

      def inner(j, off2):
        x = tmp[pl.ds(j * L, L)]
        rem = take - j * L
        m = lane < rem
        pos = [off2 + lane]
        plsc.store_scatter(asmh, pos, lax.shift_right_logical(x, 1), mask=m)
        plsc.store_scatter(aspar, pos, x & 1, mask=m)
        return off2 + jnp.minimum(rem, L)

      return lax.fori_loop(0, nv, inner, off)

    def lt_row(r, off):
      pltpu.sync_copy(cnt_sh.at[r], vec16)
      c = lane_at(vec16[...], 0)
      return copy_list(r, off, c, lt_sh)

    off = lax.fori_loop(0, NS, lt_row, jnp.int32(0))

    def eq_row(r, off):
      pltpu.sync_copy(cnt_sh.at[r], vec16)
      c = lane_at(vec16[...], 1)
      take = jnp.minimum(c, jnp.int32(NUM_PAIRS) - off)
      return copy_list(r, off, take, eq_sh)

    lax.fori_loop(0, NS, eq_row, off)
    pltpu.sync_copy(asmh, idxh_hbm.at[pl.ds(cid * NUM_PAIRS, NUM_PAIRS)])
    pltpu.sync_copy(aspar, par_hbm.at[pl.ds(cid * NUM_PAIRS, NUM_PAIRS)])


def _sc_gather_body(idxh_hbm, fmap_hbm, wide_hbm, idxv, rows, sem):
  cid = lax.axis_index("c")
  sid = lax.axis_index("s")
  base = cid * NUM_PAIRS + sid * ROWS_PER_SUB
  pltpu.sync_copy(idxh_hbm.at[pl.ds(base, ROWS_PER_SUB)], idxv)
  pltpu.async_copy(fmap_hbm.at[idxv], rows, sem).wait()
  pltpu.sync_copy(rows, wide_hbm.at[pl.ds(base, ROWS_PER_SUB)])


def _run_compact(vs_flat, prm16):
  mesh = plsc.VectorSubcoreMesh(core_axis_name="c", subcore_axis_name="s",
                                num_cores=NC, num_subcores=NS)
  fn = pl.kernel(
      _sc_compact_body,
      out_type=(
          jax.ShapeDtypeStruct((2 * NUM_PAIRS,), jnp.int32),
          jax.ShapeDtypeStruct((2 * NUM_PAIRS,), jnp.int32),
      ),
      mesh=mesh,
      scratch_types=[
          pltpu.VMEM((SHARD,), jnp.int32),
          pltpu.VMEM((SHARD,), jnp.int32),
          pltpu.VMEM((SHARD,), jnp.int32),
          pltpu.VMEM((L,), jnp.int32),
          pltpu.VMEM((NUM_PAIRS,), jnp.int32),
          pltpu.VMEM((NUM_PAIRS,), jnp.int32),
          pltpu.VMEM((SHARD,), jnp.int32),
          pltpu.VMEM_SHARED((NS, L), jnp.int32),
          pltpu.VMEM_SHARED((NS, SHARD), jnp.int32),
          pltpu.VMEM_SHARED((NS, SHARD), jnp.int32),
      ],
      compiler_params=pltpu.CompilerParams(needs_layout_passes=False,
                                           use_tc_tiling_on_sc=False),
  )
  return fn(vs_flat, prm16)


def _run_gather(idxh, fmap_pairs):
  mesh = plsc.VectorSubcoreMesh(core_axis_name="c", subcore_axis_name="s",
                                num_cores=NC, num_subcores=NS)
  fn = pl.kernel(
      _sc_gather_body,
      out_type=jax.ShapeDtypeStruct((2 * NUM_PAIRS, 128), jnp.float32),
      mesh=mesh,
      scratch_types=[
          pltpu.VMEM((ROWS_PER_SUB,), jnp.int32),
          pltpu.VMEM((ROWS_PER_SUB, 128), jnp.float32),
          pltpu.SemaphoreType.DMA,
      ],
      compiler_params=pltpu.CompilerParams(needs_layout_passes=False,
                                           use_tc_tiling_on_sc=True),
  )
  return fn(idxh, fmap_pairs)


def _desc_key_nonneg(d):
  return ~(lax.bitcast_convert_type(d, jnp.uint32) & jnp.uint32(0x7FFFFFFF))


def _topk_mean_kernel(w_ref, par_ref, out_ref):
  w1 = w_ref[0]
  w2 = w_ref[1]
  p1 = par_ref[0]
  p2 = par_ref[1]
  c1 = w1[:, 0:D_FEAT] * (1.0 - p1) + w1[:, D_FEAT:128] * p1
  c2 = w2[:, 0:D_FEAT] * (1.0 - p2) + w2[:, D_FEAT:128] * p2
  n1 = jnp.sum(c1 * c1, axis=1, keepdims=True)
  n2 = jnp.sum(c2 * c2, axis=1, keepdims=True).reshape(1, NUM_PAIRS)
  g = lax.dot_general(c1, c2, (((1,), (1,)), ((), ())),
                      preferred_element_type=jnp.float32)
  rm = jnp.max(jnp.maximum(n1 + n2 - 2.0 * g, 0.0), axis=1, keepdims=True)
  rm_row = rm.reshape(1, NUM_PAIRS)
  rk = _desc_key_nonneg(rm_row)
  t0, _ = _kth_smallest_u32(rk, TO_PICK)
  sel = rk < t0
  self_ = jnp.where(sel, 1.0, 0.0)
  tri = jnp.where(
      lax.broadcasted_iota(jnp.int32, (NUM_PAIRS, NUM_PAIRS), 0)
      <= lax.broadcasted_iota(jnp.int32, (NUM_PAIRS, NUM_PAIRS), 1),
      1.0, 0.0)
  ranks = lax.dot_general(self_, tri, (((1,), (0,)), ((), ())),
                          preferred_element_type=jnp.float32)
  nsel = jnp.sum(self_)
  oi = lax.broadcasted_iota(jnp.int32, (TO_PICK, NUM_PAIRS), 0).astype(jnp.float32)
  pmat = jnp.where((ranks - 1.0 == oi) & sel, 1.0, 0.0)
  c1s = lax.dot_general(pmat, c1, (((1,), (0,)), ((), ())),
                        preferred_element_type=jnp.float32)
  n1s = lax.dot_general(pmat, n1, (((1,), (0,)), ((), ())),
                        preferred_element_type=jnp.float32)
  gs = lax.dot_general(c1s, c2, (((1,), (1,)), ((), ())),
                       preferred_element_type=jnp.float32)
  rowvalid = lax.broadcasted_iota(jnp.int32, (TO_PICK, 1), 0).astype(jnp.float32) < nsel
  dc = jnp.where(rowvalid, jnp.maximum(n1s + n2 - 2.0 * gs, 0.0), 0.0)
  vc = _desc_key_nonneg(dc)
  pr, clr = _kth_smallest_u32(vc, TO_PICK)
  c0 = jnp.sum(jnp.where(vc < t0, jnp.int32(1), jnp.int32(0)))
  psel = jnp.where(c0 >= TO_PICK, pr, t0)
  cl = jnp.where(c0 >= TO_PICK, clr, c0)
  thr_d2 = lax.bitcast_convert_type(~psel, jnp.float32)
  ssum = jnp.sum(jnp.where(vc < psel, jnp.sqrt(dc), 0.0))
  total = ssum + (jnp.int32(TO_PICK) - cl).astype(jnp.float32) * jnp.sqrt(thr_d2)
  out_ref[...] = jnp.full((1, 1), 0.0) + total * (1.0 / TO_PICK)


def _run_topk_mean(wide, parf):
  return pl.pallas_call(
      _topk_mean_kernel,
      out_shape=jax.ShapeDtypeStruct((1, 1), jnp.float32),
  )(wide.reshape(2, NUM_PAIRS, 128), parf)


def kernel(feature_map, mask):
  n = feature_map.shape[0]
  kc = jax.random.key(42)
  g1 = jax.random.gumbel(jax.random.fold_in(kc, 0), (n,), jnp.float32)
  g2 = jax.random.gumbel(jax.random.fold_in(kc, 1), (n,), jnp.float32)
  pad = PAD_N - n
  mask2d = jnp.pad(mask, (0, pad)).reshape(ROWS2D, 128)
  g1_2d = jnp.pad(g1, (0, pad)).reshape(ROWS2D, 128)
  g2_2d = jnp.pad(g2, (0, pad)).reshape(ROWS2D, 128)

  vs1, vs2, prm = _run_select(mask2d, g1_2d, g2_2d)
  vs_flat = jnp.concatenate([vs1.reshape(-1), vs2.reshape(-1)])
  prm16 = prm.reshape(-1)[:L]

  fmap_pairs = feature_map.reshape(N_ROWS // 2, 2 * D_FEAT)
  idxh, par = _run_compact(vs_flat, prm16)
  wide = _run_gather(idxh, fmap_pairs)
  parf = par.astype(jnp.float32).reshape(2, NUM_PAIRS, 1)
  out = _run_topk_mean(wide, parf)
  return out[0, 0]
